# Initial kernel scaffold; baseline (speedup 1.0000x reference)
import jax
import jax.numpy as jnp
from jax.experimental import pallas as pl


def kernel(features, positions, edge_index_adj, node_degree_adj, W1, b1, W2, b2, P1, pb1, P2, pb2, U1, ub1, U2, ub2):
    raise NotImplementedError("write your pallas kernel here")


# speedup vs baseline: 2.9985x; 2.9985x over previous
import jax
import jax.numpy as jnp
from jax import lax
from jax.experimental import pallas as pl
from jax.experimental.pallas import tpu as pltpu
from jax.experimental.pallas import tpu_sc as plsc

_N = 10000
_E = 320000
_H = 128
_PW = 16
_PWG = 128
_NPAD = 10240
_CH = 80
_NCHUNKS = _E // _CH
_NC = 2
_NS = 16
_NW = _NC * _NS
_ITERS = _NCHUNKS // _NW
_STRIPE = _NPAD // _NS
_QROWS = _NPAD // 8
_QSTRIPE = _QROWS // _NS


def _mesh():
    return plsc.VectorSubcoreMesh(core_axis_name="c", subcore_axis_name="s")


def _sc_gather_body(tf, tp, src2d, dst2d, hs_out, hd_out, ps_out, pd_out,
                    idx_s, idx_d, rf_s, rf_d, rp_s, rp_d,
                    sem_a, sem_b, sem_c, sem_d):
    c = lax.axis_index("c")
    s = lax.axis_index("s")
    wid = s * _NC + c

    def body(i, carry):
        chunk = wid + i * _NW
        pltpu.sync_copy(src2d.at[chunk], idx_s)
        pltpu.sync_copy(dst2d.at[chunk], idx_d)
        cp1 = pltpu.async_copy(tf.at[idx_s], rf_s, sem_a)
        cp2 = pltpu.async_copy(tf.at[idx_d], rf_d, sem_b)
        cp3 = pltpu.async_copy(tp.at[idx_s], rp_s, sem_c)
        cp4 = pltpu.async_copy(tp.at[idx_d], rp_d, sem_d)
        cp1.wait()
        cp2.wait()
        cp3.wait()
        cp4.wait()
        base = chunk * _CH
        pltpu.sync_copy(rf_s, hs_out.at[pl.ds(base, _CH)])
        pltpu.sync_copy(rf_d, hd_out.at[pl.ds(base, _CH)])
        pltpu.sync_copy(rp_s, ps_out.at[pl.ds(base, _CH)])
        pltpu.sync_copy(rp_d, pd_out.at[pl.ds(base, _CH)])
        return carry

    lax.fori_loop(0, _ITERS, body, None)


def _sc_gather(tf, tp, src2d, dst2d):
    return pl.kernel(
        _sc_gather_body,
        out_type=(
            jax.ShapeDtypeStruct((_E, _H), jnp.float32),
            jax.ShapeDtypeStruct((_E, _H), jnp.float32),
            jax.ShapeDtypeStruct((_E, _PWG), jnp.float32),
            jax.ShapeDtypeStruct((_E, _PWG), jnp.float32),
        ),
        mesh=_mesh(),
        scratch_types=[
            pltpu.VMEM((_CH,), jnp.int32),
            pltpu.VMEM((_CH,), jnp.int32),
            pltpu.VMEM((_CH, _H), jnp.float32),
            pltpu.VMEM((_CH, _H), jnp.float32),
            pltpu.VMEM((_CH, _PWG), jnp.float32),
            pltpu.VMEM((_CH, _PWG), jnp.float32),
            pltpu.SemaphoreType.DMA,
            pltpu.SemaphoreType.DMA,
            pltpu.SemaphoreType.DMA,
            pltpu.SemaphoreType.DMA,
        ],
    )(tf, tp, src2d, dst2d)


def _sc_scatter_body(pay_m, pay_q, dst2d, dsthi2d, out_m, out_q,
                     idx_v, idx_h, upd_m, upd_q, acc_m, acc_q, sem):
    c = lax.axis_index("c")
    s = lax.axis_index("s")
    wid = s * _NC + c

    def zrow(i, carry):
        for j in range(_H // 16):
            upd_m[i, pl.ds(j * 16, 16)] = jnp.zeros((16,), jnp.float32)
        return carry

    lax.fori_loop(0, _CH, zrow, None)

    def zcp(r, carry):
        pltpu.sync_copy(upd_m, acc_m.at[pl.ds(s * _STRIPE + r * _CH, _CH)])
        return carry

    lax.fori_loop(0, _STRIPE // _CH, zcp, None)
    pltpu.sync_copy(upd_m, acc_q.at[pl.ds(s * _QSTRIPE, _QSTRIPE)])
    plsc.subcore_barrier()

    def body(i, carry):
        chunk = wid + i * _NW
        pltpu.sync_copy(dst2d.at[chunk], idx_v)
        pltpu.sync_copy(dsthi2d.at[chunk], idx_h)
        pltpu.sync_copy(pay_m.at[pl.ds(chunk * _CH, _CH)], upd_m)
        pltpu.sync_copy(pay_q.at[pl.ds(chunk * _CH, _CH)], upd_q)
        pltpu.sync_copy(upd_m, acc_m.at[idx_v], add=True)
        pltpu.sync_copy(upd_q, acc_q.at[idx_h], add=True)
        return carry

    lax.fori_loop(0, _ITERS, body, None)
    plsc.subcore_barrier()
    pltpu.sync_copy(acc_m.at[pl.ds(s * _STRIPE, _STRIPE)],
                    out_m.at[pl.ds(c * _NPAD + s * _STRIPE, _STRIPE)])
    pltpu.sync_copy(acc_q.at[pl.ds(s * _QSTRIPE, _QSTRIPE)],
                    out_q.at[pl.ds(c * _QROWS + s * _QSTRIPE, _QSTRIPE)])


def _sc_scatter(pay_m, pay_q, dst2d, dsthi2d):
    return pl.kernel(
        _sc_scatter_body,
        out_type=(
            jax.ShapeDtypeStruct((_NC * _NPAD, _H), jnp.float32),
            jax.ShapeDtypeStruct((_NC * _QROWS, _H), jnp.float32),
        ),
        mesh=_mesh(),
        scratch_types=[
            pltpu.VMEM((_CH,), jnp.int32),
            pltpu.VMEM((_CH,), jnp.int32),
            pltpu.VMEM((_CH, _H), jnp.float32),
            pltpu.VMEM((_CH, _H), jnp.float32),
            pltpu.VMEM_SHARED((_NPAD, _H), jnp.float32),
            pltpu.VMEM_SHARED((_QROWS, _H), jnp.float32),
            pltpu.SemaphoreType.DMA,
        ],
    )(pay_m, pay_q, dst2d, dsthi2d)


_BE = 2000


def _edge_body(hs, hd, ps, pd, dm, w1a, w1b, w1d, b1, w2, b2, p1, pb1, p2, pb2,
               out_m, out_q):
    rel = ps[:] - pd[:]
    rel16 = rel[:, :_PW]
    dist = jnp.sqrt(jnp.sum(rel16 * rel16, axis=1, keepdims=True))
    x = (jnp.dot(hs[:], w1a[:], preferred_element_type=jnp.float32)
         + jnp.dot(hd[:], w1b[:], preferred_element_type=jnp.float32)
         + dist * w1d[:] + b1[:])
    x = x * jax.nn.sigmoid(x)
    msg = jnp.dot(x, w2[:], preferred_element_type=jnp.float32) + b2[:]
    t = jnp.dot(msg, p1[:], preferred_element_type=jnp.float32) + pb1[:]
    t = t * jax.nn.sigmoid(t)
    wv = jnp.tanh(jnp.dot(t, p2[:], preferred_element_type=jnp.float32)
                  + pb2[:])
    out_m[:, :] = msg
    grp = lax.broadcasted_iota(jnp.int32, (_BE, _H), 1) >> 4
    sel = (grp == dm[:]).astype(jnp.float32)
    out_q[:, :] = wv * rel * sel


def _edge_mlp(hs, hd, ps, pd, dm, w1a, w1b, w1d, b1, w2, b2, p1, pb1, p2, pb2):
    full = lambda shp: pl.BlockSpec(shp, lambda i: (0, 0))
    return pl.pallas_call(
        _edge_body,
        grid=(_E // _BE,),
        in_specs=[
            pl.BlockSpec((_BE, _H), lambda i: (i, 0)),
            pl.BlockSpec((_BE, _H), lambda i: (i, 0)),
            pl.BlockSpec((_BE, _PWG), lambda i: (i, 0)),
            pl.BlockSpec((_BE, _PWG), lambda i: (i, 0)),
            pl.BlockSpec((_BE, 1), lambda i: (i, 0)),
            full((_H, _H)), full((_H, _H)), full((1, _H)), full((1, _H)),
            full((_H, _H)), full((1, _H)),
            full((_H, _H)), full((1, _H)),
            full((_H, 1)), full((1, 1)),
        ],
        out_specs=[
            pl.BlockSpec((_BE, _H), lambda i: (i, 0)),
            pl.BlockSpec((_BE, _H), lambda i: (i, 0)),
        ],
        out_shape=[
            jax.ShapeDtypeStruct((_E, _H), jnp.float32),
            jax.ShapeDtypeStruct((_E, _H), jnp.float32),
        ],
    )(hs, hd, ps, pd, dm, w1a, w1b, w1d, b1, w2, b2, p1, pb1, p2, pb2)


_BN = 1000


def _node_body(f, pos, a0m, a1m, a0q, a1q, deg, u1a, u1b, ub1, u2, ub2,
               fout, pout):
    aggm = a0m[:] + a1m[:]
    nrm = jnp.maximum(deg[:], 1.0)
    m = aggm / nrm
    x = (jnp.dot(f[:], u1a[:], preferred_element_type=jnp.float32)
         + jnp.dot(m, u1b[:], preferred_element_type=jnp.float32) + ub1[:])
    x = x * jax.nn.sigmoid(x)
    fu = jnp.dot(x, u2[:], preferred_element_type=jnp.float32) + ub2[:]
    fout[:, :] = f[:] + fu
    pout[:, :] = pos[:] + (a0q[:] + a1q[:])[:, :3]


def _node_mlp(f, pos, a0m, a1m, a0q, a1q, deg, u1a, u1b, ub1, u2, ub2):
    full = lambda shp: pl.BlockSpec(shp, lambda i: (0, 0))
    return pl.pallas_call(
        _node_body,
        grid=(_N // _BN,),
        in_specs=[
            pl.BlockSpec((_BN, _H), lambda i: (i, 0)),
            pl.BlockSpec((_BN, 3), lambda i: (i, 0)),
            pl.BlockSpec((_BN, _H), lambda i: (i, 0)),
            pl.BlockSpec((_BN, _H), lambda i: (i, 0)),
            pl.BlockSpec((_BN, _PW), lambda i: (i, 0)),
            pl.BlockSpec((_BN, _PW), lambda i: (i, 0)),
            pl.BlockSpec((_BN, 1), lambda i: (i, 0)),
            full((_H, _H)), full((_H, _H)), full((1, _H)),
            full((_H, _H)), full((1, _H)),
        ],
        out_specs=[
            pl.BlockSpec((_BN, _H), lambda i: (i, 0)),
            pl.BlockSpec((_BN, 3), lambda i: (i, 0)),
        ],
        out_shape=[
            jax.ShapeDtypeStruct((_N, _H), jnp.float32),
            jax.ShapeDtypeStruct((_N, 3), jnp.float32),
        ],
    )(f, pos, a0m, a1m, a0q, a1q, deg, u1a, u1b, ub1, u2, ub2)


def kernel(features, positions, edge_index_adj, node_degree_adj,
           W1, b1, W2, b2, P1, pb1, P2, pb2, U1, ub1, U2, ub2):
    f = features[0]
    pos = positions[0]
    src = edge_index_adj[0].astype(jnp.int32)
    dst = edge_index_adj[1].astype(jnp.int32)
    src2d = src.reshape(_NCHUNKS, _CH)
    dst2d = dst.reshape(_NCHUNKS, _CH)

    pos_tiled = jnp.tile(jnp.pad(pos, ((0, 0), (0, _PW - 3))), (1, 8))
    dsthi2d = (dst >> 3).reshape(_NCHUNKS, _CH)
    dstmod = (dst & 7).reshape(_E, 1)

    hs, hd, ps, pd = _sc_gather(f, pos_tiled, src2d, dst2d)

    pay_m, pay_q = _edge_mlp(
        hs, hd, ps, pd, dstmod,
        W1[:_H], W1[_H:2 * _H], W1[2 * _H:2 * _H + 1], b1.reshape(1, _H),
        W2, b2.reshape(1, _H),
        P1, pb1.reshape(1, _H),
        P2, pb2.reshape(1, 1),
    )

    agg_m, agg_q2 = _sc_scatter(pay_m, pay_q, dst2d, dsthi2d)
    agg_q = agg_q2.reshape(_NC, _NPAD, _PW)

    fout, pout = _node_mlp(
        f, pos,
        agg_m[:_N], agg_m[_NPAD:_NPAD + _N],
        agg_q[0, :_N], agg_q[1, :_N],
        node_degree_adj.reshape(_N, 1),
        U1[:_H], U1[_H:], ub1.reshape(1, _H),
        U2, ub2.reshape(1, _H),
    )
    return fout[None], pout[None]


# speedup vs baseline: 3.5036x; 1.1685x over previous
import jax
import jax.numpy as jnp
from jax import lax
from jax.experimental import pallas as pl
from jax.experimental.pallas import tpu as pltpu
from jax.experimental.pallas import tpu_sc as plsc

_N = 10000
_E = 320000
_H = 128
_HH = 64
_PW = 16
_NPAD = 10240
_CH = 80
_NCHUNKS = _E // _CH
_NC = 2
_NS = 16
_NW = _NC * _NS
_ITERS = _NCHUNKS // _NW
_STRIPE = _NPAD // _NS
_QROWS = _NPAD // 8
_QSTRIPE = _QROWS // _NS


def _mesh():
    return plsc.VectorSubcoreMesh(core_axis_name="c", subcore_axis_name="s")


_BN = 1000


def _pack_pair(x_lo, x_hi):
    lo = lax.convert_element_type(
        lax.bitcast_convert_type(x_lo.astype(jnp.bfloat16), jnp.uint16),
        jnp.uint32)
    hi = lax.convert_element_type(
        lax.bitcast_convert_type(x_hi.astype(jnp.bfloat16), jnp.uint16),
        jnp.uint32)
    return lo | (hi << 16)


def _pack_body(f, pt, w1a, w1b, b1, t1, t2):
    a = jnp.dot(f[:], w1a[:], preferred_element_type=jnp.float32) + b1[:]
    b = jnp.dot(f[:], w1b[:], preferred_element_type=jnp.float32)
    ptv = pt[:]
    ppk = _pack_pair(ptv[:, :_HH], ptv[:, _HH:])
    t1[:, :_HH] = _pack_pair(a[:, :_HH], a[:, _HH:])
    t1[:, _HH:] = ppk
    t2[:, :_HH] = _pack_pair(b[:, :_HH], b[:, _HH:])
    t2[:, _HH:] = ppk


def _pack_tables(f, pt, w1a, w1b, b1):
    full = lambda shp: pl.BlockSpec(shp, lambda i: (0, 0))
    return pl.pallas_call(
        _pack_body,
        grid=(_N // _BN,),
        in_specs=[
            pl.BlockSpec((_BN, _H), lambda i: (i, 0)),
            pl.BlockSpec((_BN, _H), lambda i: (i, 0)),
            full((_H, _H)), full((_H, _H)), full((1, _H)),
        ],
        out_specs=[
            pl.BlockSpec((_BN, _H), lambda i: (i, 0)),
            pl.BlockSpec((_BN, _H), lambda i: (i, 0)),
        ],
        out_shape=[
            jax.ShapeDtypeStruct((_N, _H), jnp.uint32),
            jax.ShapeDtypeStruct((_N, _H), jnp.uint32),
        ],
    )(f, pt, w1a, w1b, b1)


def _sc_gather_body(t1, t2, src2d, dst2d, hs_out, hd_out,
                    idx_s, idx_d, rf_s, rf_d, sem_a, sem_b):
    c = lax.axis_index("c")
    s = lax.axis_index("s")
    wid = s * _NC + c

    def body(i, carry):
        chunk = wid + i * _NW
        pltpu.sync_copy(src2d.at[chunk], idx_s)
        pltpu.sync_copy(dst2d.at[chunk], idx_d)
        cp1 = pltpu.async_copy(t1.at[idx_s], rf_s, sem_a)
        cp2 = pltpu.async_copy(t2.at[idx_d], rf_d, sem_b)
        cp1.wait()
        cp2.wait()
        base = chunk * _CH
        pltpu.sync_copy(rf_s, hs_out.at[pl.ds(base, _CH)])
        pltpu.sync_copy(rf_d, hd_out.at[pl.ds(base, _CH)])
        return carry

    lax.fori_loop(0, _ITERS, body, None)


def _sc_gather(t1, t2, src2d, dst2d):
    return pl.kernel(
        _sc_gather_body,
        out_type=(
            jax.ShapeDtypeStruct((_E, _H), jnp.uint32),
            jax.ShapeDtypeStruct((_E, _H), jnp.uint32),
        ),
        mesh=_mesh(),
        scratch_types=[
            pltpu.VMEM((_CH,), jnp.int32),
            pltpu.VMEM((_CH,), jnp.int32),
            pltpu.VMEM((_CH, _H), jnp.uint32),
            pltpu.VMEM((_CH, _H), jnp.uint32),
            pltpu.SemaphoreType.DMA,
            pltpu.SemaphoreType.DMA,
        ],
    )(t1, t2, src2d, dst2d)


def _sc_scatter_body(pay_m, pay_q, dst2d, dsthi2d, out_m, out_q,
                     idx_v, idx_h, upd_m, upd_q, acc_m, acc_q, sem):
    c = lax.axis_index("c")
    s = lax.axis_index("s")
    wid = s * _NC + c

    def zrow(i, carry):
        for j in range(_H // 16):
            upd_m[i, pl.ds(j * 16, 16)] = jnp.zeros((16,), jnp.float32)
        return carry

    lax.fori_loop(0, _CH, zrow, None)

    def zcp(r, carry):
        pltpu.sync_copy(upd_m, acc_m.at[pl.ds(s * _STRIPE + r * _CH, _CH)])
        return carry

    lax.fori_loop(0, _STRIPE // _CH, zcp, None)
    pltpu.sync_copy(upd_m, acc_q.at[pl.ds(s * _QSTRIPE, _QSTRIPE)])
    plsc.subcore_barrier()

    def body(i, carry):
        chunk = wid + i * _NW
        pltpu.sync_copy(dst2d.at[chunk], idx_v)
        pltpu.sync_copy(dsthi2d.at[chunk], idx_h)
        pltpu.sync_copy(pay_m.at[pl.ds(chunk * _CH, _CH)], upd_m)
        pltpu.sync_copy(pay_q.at[pl.ds(chunk * _CH, _CH)], upd_q)
        pltpu.sync_copy(upd_m, acc_m.at[idx_v], add=True)
        pltpu.sync_copy(upd_q, acc_q.at[idx_h], add=True)
        return carry

    lax.fori_loop(0, _ITERS, body, None)
    plsc.subcore_barrier()
    pltpu.sync_copy(acc_m.at[pl.ds(s * _STRIPE, _STRIPE)],
                    out_m.at[pl.ds(c * _NPAD + s * _STRIPE, _STRIPE)])
    pltpu.sync_copy(acc_q.at[pl.ds(s * _QSTRIPE, _QSTRIPE)],
                    out_q.at[pl.ds(c * _QROWS + s * _QSTRIPE, _QSTRIPE)])


def _sc_scatter(pay_m, pay_q, dst2d, dsthi2d):
    return pl.kernel(
        _sc_scatter_body,
        out_type=(
            jax.ShapeDtypeStruct((_NC * _NPAD, _H), jnp.float32),
            jax.ShapeDtypeStruct((_NC * _QROWS, _H), jnp.float32),
        ),
        mesh=_mesh(),
        scratch_types=[
            pltpu.VMEM((_CH,), jnp.int32),
            pltpu.VMEM((_CH,), jnp.int32),
            pltpu.VMEM((_CH, _H), jnp.float32),
            pltpu.VMEM((_CH, _H), jnp.float32),
            pltpu.VMEM_SHARED((_NPAD, _H), jnp.float32),
            pltpu.VMEM_SHARED((_QROWS, _H), jnp.float32),
            pltpu.SemaphoreType.DMA,
        ],
    )(pay_m, pay_q, dst2d, dsthi2d)


_BE = 2000


def _unpack_pair(u):
    lo = lax.bitcast_convert_type(
        lax.convert_element_type(u & jnp.uint32(0xFFFF), jnp.uint16),
        jnp.bfloat16)
    hi = lax.bitcast_convert_type(
        lax.convert_element_type(u >> jnp.uint32(16), jnp.uint16),
        jnp.bfloat16)
    return lo, hi


def _edge_body(ts, td, dm, w1d, w2, b2, p1, pb1, p2, pb2, out_m, out_q):
    slo, shi = _unpack_pair(ts[:])
    dlo, dhi = _unpack_pair(td[:])
    xl = slo[:, :_HH].astype(jnp.float32) + dlo[:, :_HH].astype(jnp.float32)
    xh = shi[:, :_HH].astype(jnp.float32) + dhi[:, :_HH].astype(jnp.float32)
    rl = slo[:, _HH:].astype(jnp.float32) - dlo[:, _HH:].astype(jnp.float32)
    rh = shi[:, _HH:].astype(jnp.float32) - dhi[:, _HH:].astype(jnp.float32)
    x = jnp.concatenate([xl, xh], axis=1)
    rel = jnp.concatenate([rl, rh], axis=1)
    rel16 = rel[:, :_PW]
    dist = jnp.sqrt(jnp.sum(rel16 * rel16, axis=1, keepdims=True))
    x = x + dist * w1d[:]
    x = x * jax.nn.sigmoid(x)
    msg = (jnp.dot(x.astype(jnp.bfloat16), w2[:],
                   preferred_element_type=jnp.float32) + b2[:])
    t = (jnp.dot(msg.astype(jnp.bfloat16), p1[:],
                 preferred_element_type=jnp.float32) + pb1[:])
    t = t * jax.nn.sigmoid(t)
    wv = jnp.tanh(jnp.dot(t, p2[:], preferred_element_type=jnp.float32)
                  + pb2[:])
    out_m[:, :] = msg
    grp = lax.broadcasted_iota(jnp.int32, (_BE, _H), 1) >> 4
    sel = (grp == dm[:]).astype(jnp.float32)
    out_q[:, :] = wv * rel * sel


def _edge_mlp(ts, td, dm, w1d, w2, b2, p1, pb1, p2, pb2):
    full = lambda shp: pl.BlockSpec(shp, lambda i: (0, 0))
    return pl.pallas_call(
        _edge_body,
        grid=(_E // _BE,),
        in_specs=[
            pl.BlockSpec((_BE, _H), lambda i: (i, 0)),
            pl.BlockSpec((_BE, _H), lambda i: (i, 0)),
            pl.BlockSpec((_BE, 1), lambda i: (i, 0)),
            full((1, _H)),
            full((_H, _H)), full((1, _H)),
            full((_H, _H)), full((1, _H)),
            full((_H, 1)), full((1, 1)),
        ],
        out_specs=[
            pl.BlockSpec((_BE, _H), lambda i: (i, 0)),
            pl.BlockSpec((_BE, _H), lambda i: (i, 0)),
        ],
        out_shape=[
            jax.ShapeDtypeStruct((_E, _H), jnp.float32),
            jax.ShapeDtypeStruct((_E, _H), jnp.float32),
        ],
    )(ts, td, dm, w1d, w2, b2, p1, pb1, p2, pb2)


def _node_body(f, pos, a0m, a1m, a0q, a1q, deg, u1a, u1b, ub1, u2, ub2,
               fout, pout):
    aggm = a0m[:] + a1m[:]
    nrm = jnp.maximum(deg[:], 1.0)
    m = aggm / nrm
    x = (jnp.dot(f[:], u1a[:], preferred_element_type=jnp.float32)
         + jnp.dot(m, u1b[:], preferred_element_type=jnp.float32) + ub1[:])
    x = x * jax.nn.sigmoid(x)
    fu = jnp.dot(x, u2[:], preferred_element_type=jnp.float32) + ub2[:]
    fout[:, :] = f[:] + fu
    pout[:, :] = pos[:] + (a0q[:] + a1q[:])[:, :3]


def _node_mlp(f, pos, a0m, a1m, a0q, a1q, deg, u1a, u1b, ub1, u2, ub2):
    full = lambda shp: pl.BlockSpec(shp, lambda i: (0, 0))
    return pl.pallas_call(
        _node_body,
        grid=(_N // _BN,),
        in_specs=[
            pl.BlockSpec((_BN, _H), lambda i: (i, 0)),
            pl.BlockSpec((_BN, 3), lambda i: (i, 0)),
            pl.BlockSpec((_BN, _H), lambda i: (i, 0)),
            pl.BlockSpec((_BN, _H), lambda i: (i, 0)),
            pl.BlockSpec((_BN, _PW), lambda i: (i, 0)),
            pl.BlockSpec((_BN, _PW), lambda i: (i, 0)),
            pl.BlockSpec((_BN, 1), lambda i: (i, 0)),
            full((_H, _H)), full((_H, _H)), full((1, _H)),
            full((_H, _H)), full((1, _H)),
        ],
        out_specs=[
            pl.BlockSpec((_BN, _H), lambda i: (i, 0)),
            pl.BlockSpec((_BN, 3), lambda i: (i, 0)),
        ],
        out_shape=[
            jax.ShapeDtypeStruct((_N, _H), jnp.float32),
            jax.ShapeDtypeStruct((_N, 3), jnp.float32),
        ],
    )(f, pos, a0m, a1m, a0q, a1q, deg, u1a, u1b, ub1, u2, ub2)


def kernel(features, positions, edge_index_adj, node_degree_adj,
           W1, b1, W2, b2, P1, pb1, P2, pb2, U1, ub1, U2, ub2):
    f = features[0]
    pos = positions[0]
    src = edge_index_adj[0].astype(jnp.int32)
    dst = edge_index_adj[1].astype(jnp.int32)
    src2d = src.reshape(_NCHUNKS, _CH)
    dst2d = dst.reshape(_NCHUNKS, _CH)
    dsthi2d = (dst >> 3).reshape(_NCHUNKS, _CH)
    dstmod = (dst & 7).reshape(_E, 1)

    pos_tiled = jnp.tile(jnp.pad(pos, ((0, 0), (0, _PW - 3))), (1, 8))

    t1, t2 = _pack_tables(f, pos_tiled, W1[:_H], W1[_H:2 * _H],
                          b1.reshape(1, _H))

    hs, hd = _sc_gather(t1, t2, src2d, dst2d)

    pay_m, pay_q = _edge_mlp(
        hs, hd, dstmod,
        W1[2 * _H:2 * _H + 1],
        W2.astype(jnp.bfloat16), b2.reshape(1, _H),
        P1.astype(jnp.bfloat16), pb1.reshape(1, _H),
        P2, pb2.reshape(1, 1),
    )

    agg_m, agg_q2 = _sc_scatter(pay_m, pay_q, dst2d, dsthi2d)
    agg_q = agg_q2.reshape(_NC, _NPAD, _PW)

    fout, pout = _node_mlp(
        f, pos,
        agg_m[:_N], agg_m[_NPAD:_NPAD + _N],
        agg_q[0, :_N], agg_q[1, :_N],
        node_degree_adj.reshape(_N, 1),
        U1[:_H], U1[_H:], ub1.reshape(1, _H),
        U2, ub2.reshape(1, _H),
    )
    return fout[None], pout[None]


# speedup vs baseline: 4.8180x; 1.3751x over previous
import jax
import jax.numpy as jnp
from jax import lax
from jax.experimental import pallas as pl
from jax.experimental.pallas import tpu as pltpu
from jax.experimental.pallas import tpu_sc as plsc

_N = 10000
_E = 320000
_H = 128
_HH = 64
_PW = 16
_NPAD = 10240
_CH = 80
_NCHUNKS = _E // _CH
_NC = 2
_NS = 16
_NW = _NC * _NS
_ITERS = _NCHUNKS // _NW
_NB = 5
_OUTER = _ITERS // _NB
_STRIPE = _NPAD // _NS
_QROWS = _NPAD // 8
_QSTRIPE = _QROWS // _NS


def _mesh():
    return plsc.VectorSubcoreMesh(core_axis_name="c", subcore_axis_name="s")


_BN = 1000


def _pack_pair(x_lo, x_hi):
    lo = lax.convert_element_type(
        lax.bitcast_convert_type(x_lo.astype(jnp.bfloat16), jnp.uint16),
        jnp.uint32)
    hi = lax.convert_element_type(
        lax.bitcast_convert_type(x_hi.astype(jnp.bfloat16), jnp.uint16),
        jnp.uint32)
    return lo | (hi << 16)


def _pack_body(f, pt, w1a, w1b, b1, t1, t2):
    a = jnp.dot(f[:], w1a[:], preferred_element_type=jnp.float32) + b1[:]
    b = jnp.dot(f[:], w1b[:], preferred_element_type=jnp.float32)
    ptv = pt[:]
    ppk = _pack_pair(ptv[:, :_HH], ptv[:, _HH:])
    t1[:, :_HH] = _pack_pair(a[:, :_HH], a[:, _HH:])
    t1[:, _HH:] = ppk
    t2[:, :_HH] = _pack_pair(b[:, :_HH], b[:, _HH:])
    t2[:, _HH:] = ppk


def _pack_tables(f, pt, w1a, w1b, b1):
    full = lambda shp: pl.BlockSpec(shp, lambda i: (0, 0))
    return pl.pallas_call(
        _pack_body,
        grid=(_N // _BN,),
        in_specs=[
            pl.BlockSpec((_BN, _H), lambda i: (i, 0)),
            pl.BlockSpec((_BN, _H), lambda i: (i, 0)),
            full((_H, _H)), full((_H, _H)), full((1, _H)),
        ],
        out_specs=[
            pl.BlockSpec((_BN, _H), lambda i: (i, 0)),
            pl.BlockSpec((_BN, _H), lambda i: (i, 0)),
        ],
        out_shape=[
            jax.ShapeDtypeStruct((_N, _H), jnp.uint32),
            jax.ShapeDtypeStruct((_N, _H), jnp.uint32),
        ],
    )(f, pt, w1a, w1b, b1)


def _sc_gather_body(t1, t2, src1d, dst1d, hs_out, hd_out, idxs, idxd,
                    *bufs_and_sems):
    rfs = bufs_and_sems[:_NB]
    rfd = bufs_and_sems[_NB:2 * _NB]
    sem_s = bufs_and_sems[2 * _NB:3 * _NB]
    sem_d = bufs_and_sems[3 * _NB:4 * _NB]
    c = lax.axis_index("c")
    s = lax.axis_index("s")
    wid = s * _NC + c
    wbase = wid * _ITERS
    pltpu.sync_copy(src1d.at[pl.ds(wbase * _CH, _ITERS * _CH)], idxs)
    pltpu.sync_copy(dst1d.at[pl.ds(wbase * _CH, _ITERS * _CH)], idxd)

    def body(g, carry):
        row0 = g * _NB
        gs = [pltpu.async_copy(
                  t1.at[idxs.at[pl.ds((row0 + b) * _CH, _CH)]],
                  rfs[b], sem_s[b])
              for b in range(_NB)]
        gd = [pltpu.async_copy(
                  t2.at[idxd.at[pl.ds((row0 + b) * _CH, _CH)]],
                  rfd[b], sem_d[b])
              for b in range(_NB)]
        ws, wd = [], []
        for b in range(_NB):
            gs[b].wait()
            gd[b].wait()
            base = (wbase + row0 + b) * _CH
            ws.append(pltpu.async_copy(rfs[b], hs_out.at[pl.ds(base, _CH)],
                                       sem_s[b]))
            wd.append(pltpu.async_copy(rfd[b], hd_out.at[pl.ds(base, _CH)],
                                       sem_d[b]))
        for b in range(_NB):
            ws[b].wait()
            wd[b].wait()
        return carry

    lax.fori_loop(0, _OUTER, body, None)


def _sc_gather(t1, t2, src1d, dst1d):
    return pl.kernel(
        _sc_gather_body,
        out_type=(
            jax.ShapeDtypeStruct((_E, _H), jnp.uint32),
            jax.ShapeDtypeStruct((_E, _H), jnp.uint32),
        ),
        mesh=_mesh(),
        scratch_types=(
            [pltpu.VMEM((_ITERS * _CH,), jnp.int32)] * 2
            + [pltpu.VMEM((_CH, _H), jnp.uint32)] * (2 * _NB)
            + [pltpu.SemaphoreType.DMA] * (2 * _NB)
        ),
    )(t1, t2, src1d, dst1d)


def _sc_scatter_body(pay_m, pay_q, dst1d, dsthi1d, out_m, out_q,
                     acc_m, acc_q, *bufs_and_sems):
    _NBS = 3
    pb = bufs_and_sems[:_NBS]
    ix = bufs_and_sems[_NBS:2 * _NBS]
    sem_p = bufs_and_sems[2 * _NBS:3 * _NBS]
    sem_i = bufs_and_sems[3 * _NBS:4 * _NBS]
    c = lax.axis_index("c")
    s = lax.axis_index("s")
    wid = s * _NC + c
    wbase = wid * _ITERS

    def zrow(i, carry):
        for j in range(_H // 16):
            pb[0][i, pl.ds(j * 16, 16)] = jnp.zeros((16,), jnp.float32)
        return carry

    lax.fori_loop(0, _CH, zrow, None)

    def zcp(r, carry):
        pltpu.sync_copy(pb[0], acc_m.at[pl.ds(s * _STRIPE + r * _CH, _CH)])
        return carry

    lax.fori_loop(0, _STRIPE // _CH, zcp, None)
    pltpu.sync_copy(pb[0], acc_q.at[pl.ds(s * _QSTRIPE, _QSTRIPE)])
    plsc.subcore_barrier()

    def run_phase(pay, idx1d, acc):
        def group(row0, nb):
            ri, rp = [], []
            for b in range(nb):
                base = (wbase + row0 + b) * _CH
                ri.append(pltpu.async_copy(idx1d.at[pl.ds(base, _CH)],
                                           ix[b], sem_i[b]))
                rp.append(pltpu.async_copy(pay.at[pl.ds(base, _CH)],
                                           pb[b], sem_p[b]))
            sc = []
            for b in range(nb):
                ri[b].wait()
                rp[b].wait()
                sc.append(pltpu.async_copy(pb[b], acc.at[ix[b]],
                                           sem_p[b], add=True))
            for cp in sc:
                cp.wait()

        def body(g, carry):
            group(g * _NBS, _NBS)
            return carry

        nfull = _ITERS // _NBS
        lax.fori_loop(0, nfull, body, None)
        if _ITERS % _NBS:
            group(nfull * _NBS, _ITERS % _NBS)

    run_phase(pay_m, dst1d, acc_m)
    run_phase(pay_q, dsthi1d, acc_q)
    plsc.subcore_barrier()
    pltpu.sync_copy(acc_m.at[pl.ds(s * _STRIPE, _STRIPE)],
                    out_m.at[pl.ds(c * _NPAD + s * _STRIPE, _STRIPE)])
    pltpu.sync_copy(acc_q.at[pl.ds(s * _QSTRIPE, _QSTRIPE)],
                    out_q.at[pl.ds(c * _QROWS + s * _QSTRIPE, _QSTRIPE)])


def _sc_scatter(pay_m, pay_q, dst1d, dsthi1d):
    return pl.kernel(
        _sc_scatter_body,
        out_type=(
            jax.ShapeDtypeStruct((_NC * _NPAD, _H), jnp.float32),
            jax.ShapeDtypeStruct((_NC * _QROWS, _H), jnp.float32),
        ),
        mesh=_mesh(),
        scratch_types=(
            [pltpu.VMEM_SHARED((_NPAD, _H), jnp.float32),
             pltpu.VMEM_SHARED((_QROWS, _H), jnp.float32)]
            + [pltpu.VMEM((_CH, _H), jnp.float32)] * 3
            + [pltpu.VMEM((_CH,), jnp.int32)] * 3
            + [pltpu.SemaphoreType.DMA] * 6
        ),
    )(pay_m, pay_q, dst1d, dsthi1d)


_BE = 2000


def _unpack_pair(u):
    lo = lax.bitcast_convert_type(
        lax.convert_element_type(u & jnp.uint32(0xFFFF), jnp.uint16),
        jnp.bfloat16)
    hi = lax.bitcast_convert_type(
        lax.convert_element_type(u >> jnp.uint32(16), jnp.uint16),
        jnp.bfloat16)
    return lo, hi


def _edge_body(ts, td, dm, w1d, w2, b2, p1, pb1, p2, pb2, out_m, out_q):
    slo, shi = _unpack_pair(ts[:])
    dlo, dhi = _unpack_pair(td[:])
    xl = slo[:, :_HH].astype(jnp.float32) + dlo[:, :_HH].astype(jnp.float32)
    xh = shi[:, :_HH].astype(jnp.float32) + dhi[:, :_HH].astype(jnp.float32)
    rl = slo[:, _HH:].astype(jnp.float32) - dlo[:, _HH:].astype(jnp.float32)
    rh = shi[:, _HH:].astype(jnp.float32) - dhi[:, _HH:].astype(jnp.float32)
    x = jnp.concatenate([xl, xh], axis=1)
    rel = jnp.concatenate([rl, rh], axis=1)
    rel16 = rel[:, :_PW]
    dist = jnp.sqrt(jnp.sum(rel16 * rel16, axis=1, keepdims=True))
    x = x + dist * w1d[:]
    x = x * jax.nn.sigmoid(x)
    msg = (jnp.dot(x.astype(jnp.bfloat16), w2[:],
                   preferred_element_type=jnp.float32) + b2[:])
    t = (jnp.dot(msg.astype(jnp.bfloat16), p1[:],
                 preferred_element_type=jnp.float32) + pb1[:])
    t = t * jax.nn.sigmoid(t)
    wv = jnp.tanh(jnp.dot(t, p2[:], preferred_element_type=jnp.float32)
                  + pb2[:])
    out_m[:, :] = msg
    grp = lax.broadcasted_iota(jnp.int32, (_BE, _H), 1) >> 4
    sel = (grp == dm[:]).astype(jnp.float32)
    out_q[:, :] = wv * rel * sel


def _edge_mlp(ts, td, dm, w1d, w2, b2, p1, pb1, p2, pb2):
    full = lambda shp: pl.BlockSpec(shp, lambda i: (0, 0))
    return pl.pallas_call(
        _edge_body,
        grid=(_E // _BE,),
        in_specs=[
            pl.BlockSpec((_BE, _H), lambda i: (i, 0)),
            pl.BlockSpec((_BE, _H), lambda i: (i, 0)),
            pl.BlockSpec((_BE, 1), lambda i: (i, 0)),
            full((1, _H)),
            full((_H, _H)), full((1, _H)),
            full((_H, _H)), full((1, _H)),
            full((_H, 1)), full((1, 1)),
        ],
        out_specs=[
            pl.BlockSpec((_BE, _H), lambda i: (i, 0)),
            pl.BlockSpec((_BE, _H), lambda i: (i, 0)),
        ],
        out_shape=[
            jax.ShapeDtypeStruct((_E, _H), jnp.float32),
            jax.ShapeDtypeStruct((_E, _H), jnp.float32),
        ],
    )(ts, td, dm, w1d, w2, b2, p1, pb1, p2, pb2)


def _node_body(f, pos, a0m, a1m, a0q, a1q, deg, u1a, u1b, ub1, u2, ub2,
               fout, pout):
    aggm = a0m[:] + a1m[:]
    nrm = jnp.maximum(deg[:], 1.0)
    m = aggm / nrm
    x = (jnp.dot(f[:], u1a[:], preferred_element_type=jnp.float32)
         + jnp.dot(m, u1b[:], preferred_element_type=jnp.float32) + ub1[:])
    x = x * jax.nn.sigmoid(x)
    fu = jnp.dot(x, u2[:], preferred_element_type=jnp.float32) + ub2[:]
    fout[:, :] = f[:] + fu
    pout[:, :] = pos[:] + (a0q[:] + a1q[:])[:, :3]


def _node_mlp(f, pos, a0m, a1m, a0q, a1q, deg, u1a, u1b, ub1, u2, ub2):
    full = lambda shp: pl.BlockSpec(shp, lambda i: (0, 0))
    return pl.pallas_call(
        _node_body,
        grid=(_N // _BN,),
        in_specs=[
            pl.BlockSpec((_BN, _H), lambda i: (i, 0)),
            pl.BlockSpec((_BN, 3), lambda i: (i, 0)),
            pl.BlockSpec((_BN, _H), lambda i: (i, 0)),
            pl.BlockSpec((_BN, _H), lambda i: (i, 0)),
            pl.BlockSpec((_BN, _PW), lambda i: (i, 0)),
            pl.BlockSpec((_BN, _PW), lambda i: (i, 0)),
            pl.BlockSpec((_BN, 1), lambda i: (i, 0)),
            full((_H, _H)), full((_H, _H)), full((1, _H)),
            full((_H, _H)), full((1, _H)),
        ],
        out_specs=[
            pl.BlockSpec((_BN, _H), lambda i: (i, 0)),
            pl.BlockSpec((_BN, 3), lambda i: (i, 0)),
        ],
        out_shape=[
            jax.ShapeDtypeStruct((_N, _H), jnp.float32),
            jax.ShapeDtypeStruct((_N, 3), jnp.float32),
        ],
    )(f, pos, a0m, a1m, a0q, a1q, deg, u1a, u1b, ub1, u2, ub2)


def kernel(features, positions, edge_index_adj, node_degree_adj,
           W1, b1, W2, b2, P1, pb1, P2, pb2, U1, ub1, U2, ub2):
    f = features[0]
    pos = positions[0]
    src = edge_index_adj[0].astype(jnp.int32)
    dst = edge_index_adj[1].astype(jnp.int32)
    dsthi = dst >> 3
    dstmod = (dst & 7).reshape(_E, 1)

    pos_tiled = jnp.tile(jnp.pad(pos, ((0, 0), (0, _PW - 3))), (1, 8))

    t1, t2 = _pack_tables(f, pos_tiled, W1[:_H], W1[_H:2 * _H],
                          b1.reshape(1, _H))

    hs, hd = _sc_gather(t1, t2, src, dst)

    pay_m, pay_q = _edge_mlp(
        hs, hd, dstmod,
        W1[2 * _H:2 * _H + 1],
        W2.astype(jnp.bfloat16), b2.reshape(1, _H),
        P1.astype(jnp.bfloat16), pb1.reshape(1, _H),
        P2, pb2.reshape(1, 1),
    )

    agg_m, agg_q2 = _sc_scatter(pay_m, pay_q, dst, dsthi)
    agg_q = agg_q2.reshape(_NC, _NPAD, _PW)

    fout, pout = _node_mlp(
        f, pos,
        agg_m[:_N], agg_m[_NPAD:_NPAD + _N],
        agg_q[0, :_N], agg_q[1, :_N],
        node_degree_adj.reshape(_N, 1),
        U1[:_H], U1[_H:], ub1.reshape(1, _H),
        U2, ub2.reshape(1, _H),
    )
    return fout[None], pout[None]
